# baseline (speedup 1.0000x reference)
import functools

import jax
import jax.numpy as jnp
from jax import lax
from jax.experimental import pallas as pl
from jax.experimental.pallas import tpu as pltpu

NKF = 8
BM = 256


def _dot_t(a, b, prec=None):
    return lax.dot_general(
        a, b, (((1,), (1,)), ((), ())),
        preferred_element_type=jnp.float32, precision=prec)


def _body(wf_hbm, bf_hbm, wf_tail_ref, bf_tail_ref, stm_ref, w0_ref,
          b0_ref, w1_ref, b1_ref, w2_ref, b2_ref, w3_ref, b3_ref, out_ref,
          acc_w, acc_b, xw_buf, xb_buf, sems,
          *, num_features, bk, nk, bm, nb):
    k = pl.program_id(0)
    i = pl.program_id(1)
    step = k * nb + i
    manual_steps = (nk - 1) * nb
    hm = bm // 2
    rem = num_features - (nk - 1) * bk

    def issue(k2, i2, slot):
        for s, (src, buf) in enumerate(
                ((wf_hbm, xw_buf), (wf_hbm, xw_buf),
                 (bf_hbm, xb_buf), (bf_hbm, xb_buf))):
            h = s % 2
            pltpu.make_async_copy(
                src.at[pl.ds(i2 * bm + h * hm, hm), pl.ds(k2 * bk, bk)],
                buf.at[slot, pl.ds(h * hm, hm), :],
                sems.at[slot, s],
            ).start(priority=s % 2)

    def wait(slot):
        for s, buf in enumerate((xw_buf, xw_buf, xb_buf, xb_buf)):
            h = s % 2
            pltpu.make_async_copy(
                wf_hbm.at[pl.ds(0, hm), pl.ds(0, bk)],
                buf.at[slot, pl.ds(h * hm, hm), :],
                sems.at[slot, s],
            ).wait()

    @pl.when(step == 0)
    def _prologue():
        acc_w[...] = jnp.zeros_like(acc_w)
        acc_b[...] = jnp.zeros_like(acc_b)
        issue(0, 0, 0)

    @pl.when(step + 1 < manual_steps)
    def _prefetch():
        s1 = step + 1
        issue(s1 // nb, s1 % nb, s1 % 2)

    row = pl.ds(i * bm, bm)

    @pl.when(k < nk - 1)
    def _accum_full():
        slot = step % 2
        wait(slot)
        w0 = w0_ref[...]
        acc_w[row, :] += _dot_t(xw_buf[slot], w0)
        acc_b[row, :] += _dot_t(xb_buf[slot], w0)

    @pl.when(k == nk - 1)
    def _accum_tail_and_finish():
        wt = wf_tail_ref.shape[1]
        colx = lax.broadcasted_iota(jnp.int32, wf_tail_ref.shape, 1)
        colw = lax.broadcasted_iota(jnp.int32, (w0_ref.shape[0], wt), 1)
        xw = jnp.where(colx < rem, wf_tail_ref[...], 0.0)
        xb = jnp.where(colx < rem, bf_tail_ref[...], 0.0)
        w0 = jnp.where(colw < rem, w0_ref[:, :wt], 0.0)
        w = acc_w[row, :] + _dot_t(xw, w0) + b0_ref[...]
        b = acc_b[row, :] + _dot_t(xb, w0) + b0_ref[...]
        stm = stm_ref[...]
        wb = jnp.concatenate([w, b], axis=1)
        bw = jnp.concatenate([b, w], axis=1)
        accum = stm * wb + (1.0 - stm) * bw
        l1_x = jnp.clip(accum, 0.0, 1.0)
        hi = lax.Precision.HIGHEST
        l2_x = jnp.clip(_dot_t(l1_x, w1_ref[...], hi) + b1_ref[...], 0.0, 1.0)
        l3_x = jnp.clip(_dot_t(l2_x, w2_ref[...], hi) + b2_ref[...], 0.0, 1.0)
        out_ref[...] = (jnp.sum(l3_x * w3_ref[...], axis=1, keepdims=True)
                        + b3_ref[0, 0])


def kernel(white_features, black_features, stm, l0_w, l0_b, l1_w, l1_b,
           l2_w, l2_b, l3_w, l3_b):
    B, F = white_features.shape
    M = l0_w.shape[0]
    bm = min(BM, B)
    bk = (F // (NKF * 128)) * 128
    nkf = F // bk
    rem = F - nkf * bk
    wt = 128
    nk = nkf + 1
    nb = B // bm
    last = nk - 1

    def tail_map(k, i):
        return (jnp.where(k == last, i, 0), (nkf * bk) // wt)

    body = functools.partial(_body, num_features=F, bk=bk, nk=nk, bm=bm,
                             nb=nb)
    out = pl.pallas_call(
        body,
        grid=(nk, nb),
        in_specs=[
            pl.BlockSpec(memory_space=pl.ANY),
            pl.BlockSpec(memory_space=pl.ANY),
            pl.BlockSpec((bm, wt), tail_map),
            pl.BlockSpec((bm, wt), tail_map),
            pl.BlockSpec((bm, 2 * M),
                         lambda k, i: (jnp.where(k == last, i, 0), 0)),
            pl.BlockSpec((M, bk), lambda k, i: (0, k)),
            pl.BlockSpec((1, M), lambda k, i: (0, 0)),
            pl.BlockSpec(l1_w.shape, lambda k, i: (0, 0)),
            pl.BlockSpec((1, l1_w.shape[0]), lambda k, i: (0, 0)),
            pl.BlockSpec(l2_w.shape, lambda k, i: (0, 0)),
            pl.BlockSpec((1, l2_w.shape[0]), lambda k, i: (0, 0)),
            pl.BlockSpec(l3_w.shape, lambda k, i: (0, 0)),
            pl.BlockSpec(memory_space=pltpu.SMEM),
        ],
        out_specs=pl.BlockSpec((bm, l3_w.shape[0]),
                               lambda k, i: (jnp.where(k == last, i, 0), 0)),
        out_shape=jax.ShapeDtypeStruct((B, l3_w.shape[0]), jnp.float32),
        scratch_shapes=[
            pltpu.VMEM((B, M), jnp.float32),
            pltpu.VMEM((B, M), jnp.float32),
            pltpu.VMEM((2, bm, bk), jnp.float32),
            pltpu.VMEM((2, bm, bk), jnp.float32),
            pltpu.SemaphoreType.DMA((2, 4)),
        ],
        compiler_params=pltpu.CompilerParams(
            dimension_semantics=("arbitrary", "arbitrary"),
        ),
    )(white_features, black_features, white_features, black_features,
      stm, l0_w, l0_b.reshape(1, -1), l1_w, l1_b.reshape(1, -1),
      l2_w, l2_b.reshape(1, -1), l3_w, l3_b.reshape(1, -1))
    return out
